# baseline (speedup 1.0000x reference)
import functools

import jax
import jax.numpy as jnp
import numpy as np
from jax import lax
from jax.experimental import pallas as pl
from jax.experimental.pallas import tpu as pltpu
from jax.experimental.pallas import tpu_sc as plsc

B, T, C_IN = 1024, 200, 131
C_OUT = 84
ROWS = B * T

_COLS = np.array(
    [i for i in range(3, 66) if (i - 3) % 3 != 2]
    + [i for i in range(68, 131) if (i - 68) % 3 != 2],
    dtype=np.int32,
)
assert _COLS.shape[0] == C_OUT

GROUP_OUT = 336
N_PAT = GROUP_OUT // 16
_IDX_NP = np.array(
    [(p // C_OUT) * C_IN + _COLS[p % C_OUT] for p in range(GROUP_OUT)],
    dtype=np.int32,
)

NC = 2
NS = 16
NW = NC * NS
ROWS_PER_W = ROWS // NW
R = 160
STEPS = ROWS_PER_W // R
PAIRS = STEPS // 2
GROUPS = R // 4
IN_CHUNK = R * C_IN
OUT_CHUNK = R * C_OUT


def _sc_reduce(x_flat, idx):
    mesh = plsc.VectorSubcoreMesh(core_axis_name="c", subcore_axis_name="s")

    @functools.partial(
        pl.kernel,
        mesh=mesh,
        out_type=jax.ShapeDtypeStruct((ROWS * C_OUT,), jnp.float32),
        scratch_types=[
            pltpu.VMEM((GROUP_OUT,), jnp.int32),
            pltpu.VMEM((IN_CHUNK,), jnp.float32),
            pltpu.VMEM((OUT_CHUNK,), jnp.float32),
            pltpu.VMEM_SHARED((2, NS, IN_CHUNK), jnp.float32),
            pltpu.VMEM_SHARED((2, NS, OUT_CHUNK), jnp.float32),
            pltpu.SemaphoreType.DMA,
            pltpu.SemaphoreType.DMA,
            pltpu.SemaphoreType.DMA,
            pltpu.SemaphoreType.DMA,
        ],
        compiler_params=pltpu.CompilerParams(
            needs_layout_passes=False, use_tc_tiling_on_sc=False),
    )
    def k(x_hbm, idx_hbm, out_hbm, idx_v, in_v, out_v, smem_in, smem_out,
          sa0, sa1, se0, se1):
        sid = lax.axis_index("s")
        wid = sid * NC + lax.axis_index("c")
        in_base = wid * (ROWS_PER_W * C_IN)
        out_base = wid * (ROWS_PER_W * C_OUT)
        pltpu.sync_copy(idx_hbm, idx_v)
        pats = [idx_v[pl.ds(j * 16, 16)] for j in range(N_PAT)]
        sas = (sa0, sa1)
        ses = (se0, se1)

        def start_a(s, p):
            s = jnp.minimum(s, STEPS - 1)
            pltpu.async_copy(
                x_hbm.at[pl.ds(in_base + s * IN_CHUNK, IN_CHUNK)],
                smem_in.at[p, sid], sas[p])

        def wait_a(p):
            pltpu.make_async_copy(
                x_hbm.at[pl.ds(0, IN_CHUNK)], smem_in.at[p, sid],
                sas[p]).wait()

        def start_e(s, p):
            pltpu.async_copy(
                smem_out.at[p, sid],
                out_hbm.at[pl.ds(out_base + s * OUT_CHUNK, OUT_CHUNK)],
                ses[p])

        def wait_e(p):
            pltpu.make_async_copy(
                smem_out.at[p, sid], out_hbm.at[pl.ds(0, OUT_CHUNK)],
                ses[p]).wait()

        def compute():
            def grp(g, c):
                base = jnp.full((16,), g * (4 * C_IN), jnp.int32)
                for j in range(N_PAT):
                    out_v[pl.ds(g * GROUP_OUT + j * 16, 16)] = (
                        plsc.load_gather(in_v, [pats[j] + base]))
                return c
            lax.fori_loop(0, GROUPS, grp, 0)

        def step(s, p, first):
            wait_a(p)
            pltpu.sync_copy(smem_in.at[p, sid], in_v)
            start_a(s + 2, p)
            compute()
            if not first:
                wait_e(p)
            pltpu.sync_copy(out_v, smem_out.at[p, sid])
            start_e(s, p)

        start_a(0, 0)
        start_a(1, 1)
        step(0, 0, True)
        step(1, 1, True)

        def pair(t, c):
            step(2 * t, 0, False)
            step(2 * t + 1, 1, False)
            return c

        lax.fori_loop(1, PAIRS, pair, 0)

        wait_a(0)
        wait_a(1)
        wait_e(0)
        wait_e(1)

    return k(x_flat, idx)


def kernel(X):
    x_flat = X.reshape(-1)
    idx = jnp.asarray(_IDX_NP)
    out_flat = _sc_reduce(x_flat, idx)
    return out_flat.reshape(B, T, C_OUT)


# speedup vs baseline: 1.0875x; 1.0875x over previous
import functools

import jax
import jax.numpy as jnp
import numpy as np
from jax import lax
from jax.experimental import pallas as pl
from jax.experimental.pallas import tpu as pltpu
from jax.experimental.pallas import tpu_sc as plsc

B, T, C_IN = 1024, 200, 131
C_OUT = 84

_COLS = np.array(
    [i for i in range(3, 66) if (i - 3) % 3 != 2]
    + [i for i in range(68, 131) if (i - 68) % 3 != 2],
    dtype=np.int32,
)
assert _COLS.shape[0] == C_OUT

GROUP_OUT = 336
N_PAT = GROUP_OUT // 16
_IDX_NP = np.array(
    [(p // C_OUT) * C_IN + _COLS[p % C_OUT] for p in range(GROUP_OUT)],
    dtype=np.int32,
)

NC = 2
NS = 16
NW = NC * NS
B_PER_W = B // NW
GROUPS = T // 4
IN_CHUNK = T * C_IN
OUT_CHUNK = T * C_OUT
PAIRS = B_PER_W // 2


def _sc_reduce(x2, idx):
    mesh = plsc.VectorSubcoreMesh(core_axis_name="c", subcore_axis_name="s")

    @functools.partial(
        pl.kernel,
        mesh=mesh,
        out_type=jax.ShapeDtypeStruct((B, OUT_CHUNK), jnp.float32),
        scratch_types=[
            pltpu.VMEM((GROUP_OUT,), jnp.int32),
            pltpu.VMEM((IN_CHUNK,), jnp.float32),
            pltpu.VMEM((IN_CHUNK,), jnp.float32),
            pltpu.VMEM((OUT_CHUNK,), jnp.float32),
            pltpu.VMEM((OUT_CHUNK,), jnp.float32),
            pltpu.SemaphoreType.DMA,
            pltpu.SemaphoreType.DMA,
            pltpu.SemaphoreType.DMA,
            pltpu.SemaphoreType.DMA,
        ],
        compiler_params=pltpu.CompilerParams(
            needs_layout_passes=False, use_tc_tiling_on_sc=False),
    )
    def k(x_hbm, idx_hbm, out_hbm, idx_v, in0, in1, out0, out1,
          sin0, sin1, sout0, sout1):
        wid = lax.axis_index("s") * NC + lax.axis_index("c")
        b0 = wid * B_PER_W
        pltpu.sync_copy(idx_hbm, idx_v)
        pats = [idx_v[pl.ds(j * 16, 16)] for j in range(N_PAT)]

        def start_in(s, buf, sem):
            s = jnp.minimum(s, B_PER_W - 1)
            pltpu.async_copy(x_hbm.at[b0 + s], buf, sem)

        def wait_in(buf, sem):
            pltpu.make_async_copy(x_hbm.at[0], buf, sem).wait()

        def start_out(buf, s, sem):
            pltpu.async_copy(buf, out_hbm.at[b0 + s], sem)

        def wait_out(buf, sem):
            pltpu.make_async_copy(buf, out_hbm.at[0], sem).wait()

        def compute(in_ref, out_ref):
            def grp(g, c):
                base = jnp.full((16,), g * (4 * C_IN), jnp.int32)
                for j in range(N_PAT):
                    out_ref[pl.ds(g * GROUP_OUT + j * 16, 16)] = (
                        plsc.load_gather(in_ref, [pats[j] + base]))
                return c
            lax.fori_loop(0, GROUPS, grp, 0)

        start_in(0, in0, sin0)
        start_in(1, in1, sin1)
        wait_in(in0, sin0)
        compute(in0, out0)
        start_out(out0, 0, sout0)
        start_in(2, in0, sin0)
        wait_in(in1, sin1)
        compute(in1, out1)
        start_out(out1, 1, sout1)
        start_in(3, in1, sin1)

        def pair(t, c):
            s0 = 2 * t
            wait_in(in0, sin0)
            wait_out(out0, sout0)
            compute(in0, out0)
            start_out(out0, s0, sout0)
            start_in(s0 + 2, in0, sin0)
            wait_in(in1, sin1)
            wait_out(out1, sout1)
            compute(in1, out1)
            start_out(out1, s0 + 1, sout1)
            start_in(s0 + 3, in1, sin1)
            return c

        lax.fori_loop(1, PAIRS, pair, 0)

        wait_in(in0, sin0)
        wait_in(in1, sin1)
        wait_out(out0, sout0)
        wait_out(out1, sout1)

    return k(x2, idx)


def kernel(X):
    x2 = X.reshape(B, T * C_IN)
    idx = jnp.asarray(_IDX_NP)
    out2 = _sc_reduce(x2, idx)
    return out2.reshape(B, T, C_OUT)
